# baseline (speedup 1.0000x reference)
import jax
import jax.numpy as jnp
from jax.experimental import pallas as pl
from jax.experimental.pallas import tpu as pltpu

_E_TOTAL = 160000 * 16
_NBUF = 4
_ECHUNK = _E_TOTAL // 8
_NCHUNK = _E_TOTAL // _ECHUNK


def _copy_body(x_hbm, e_hbm, xo_hbm, eo_hbm,
               xbuf, eb0, eb1, eb2, eb3,
               sx_in, sx_out, si0, si1, si2, si3, so0, so1, so2, so3):
    ebufs = (eb0, eb1, eb2, eb3)
    sin = (si0, si1, si2, si3)
    sout = (so0, so1, so2, so3)

    def e_in(i):
        return pltpu.make_async_copy(
            e_hbm.at[pl.ds(i * _ECHUNK, _ECHUNK)], ebufs[i % _NBUF], sin[i % _NBUF])

    def e_out(i):
        return pltpu.make_async_copy(
            ebufs[i % _NBUF], eo_hbm.at[pl.ds(i * _ECHUNK, _ECHUNK)], sout[i % _NBUF])

    x_in = pltpu.make_async_copy(x_hbm, xbuf, sx_in)
    x_in.start()
    for i in range(_NBUF):
        e_in(i).start()
    x_in.wait()
    x_out = pltpu.make_async_copy(xbuf, xo_hbm, sx_out)
    x_out.start()
    for i in range(_NCHUNK):
        e_in(i).wait()
        e_out(i).start()
        j = i + _NBUF
        if j < _NCHUNK:
            e_out(j - _NBUF).wait()
            e_in(j).start()
    for i in range(max(_NCHUNK - _NBUF, 0), _NCHUNK):
        e_out(i).wait()
    x_out.wait()


def kernel(x, edge_index, edge_attr):
    del edge_index
    e1d = edge_attr.reshape(_E_TOTAL)
    x_out, e_out = pl.pallas_call(
        _copy_body,
        in_specs=[
            pl.BlockSpec(memory_space=pl.ANY),
            pl.BlockSpec(memory_space=pl.ANY),
        ],
        out_specs=[
            pl.BlockSpec(memory_space=pl.ANY),
            pl.BlockSpec(memory_space=pl.ANY),
        ],
        out_shape=[
            jax.ShapeDtypeStruct(x.shape, x.dtype),
            jax.ShapeDtypeStruct((_E_TOTAL,), edge_attr.dtype),
        ],
        scratch_shapes=[
            pltpu.VMEM((10000, 256), jnp.float32),
            pltpu.VMEM((_ECHUNK,), jnp.float32),
            pltpu.VMEM((_ECHUNK,), jnp.float32),
            pltpu.VMEM((_ECHUNK,), jnp.float32),
            pltpu.VMEM((_ECHUNK,), jnp.float32),
        ] + [pltpu.SemaphoreType.DMA] * 10,
    )(x, e1d)
    return (x_out, e_out.reshape(edge_attr.shape))


# speedup vs baseline: 1.2262x; 1.2262x over previous
import jax
import jax.numpy as jnp
from jax.experimental import pallas as pl

_GRID = 10
_XBLK = 1000
_EBLK = 16000


def _copy_body(x_ref, e_ref, xo_ref, eo_ref):
    xo_ref[...] = x_ref[...]
    eo_ref[...] = e_ref[...]


def kernel(x, edge_index, edge_attr):
    del edge_index
    x_out, e_out = pl.pallas_call(
        _copy_body,
        grid=(_GRID,),
        in_specs=[
            pl.BlockSpec((_XBLK, 256), lambda i: (i, 0)),
            pl.BlockSpec((_EBLK, 16), lambda i: (i, 0)),
        ],
        out_specs=[
            pl.BlockSpec((_XBLK, 256), lambda i: (i, 0)),
            pl.BlockSpec((_EBLK, 16), lambda i: (i, 0)),
        ],
        out_shape=[
            jax.ShapeDtypeStruct(x.shape, x.dtype),
            jax.ShapeDtypeStruct(edge_attr.shape, edge_attr.dtype),
        ],
    )(x, edge_attr)
    return (x_out, e_out)
